# Initial kernel scaffold; baseline (speedup 1.0000x reference)
import jax
import jax.numpy as jnp
from jax.experimental import pallas as pl


def kernel(sfeat, tfeat):
    raise NotImplementedError("write your pallas kernel here")


# speedup vs baseline: 1.5297x; 1.5297x over previous
import functools
from itertools import product

import jax
import jax.numpy as jnp
import numpy as np
from jax.experimental import pallas as pl

_B = 4096
_C = 6
_D = 512
_DOM_LEN = 64
_TMP_LEN = 720
_BS = 256


def _select_kernel(mask_ref, s_ref, t_ref, out_ref):
    m = mask_ref[...]
    s = s_ref[...]
    t = t_ref[...]
    d = m * (t - s)
    out_ref[0] = s + d
    out_ref[1] = t - d


@functools.partial(jax.jit, static_argnums=())
def _labels():
    rkey = jax.random.key(1234)
    ka, kb = jax.random.split(rkey)
    tem_rand_lab = jax.random.randint(ka, (_B,), 0, _TMP_LEN)
    dom_rand_lab1 = jax.random.randint(kb, (_B,), 0, _DOM_LEN // 2)
    return tem_rand_lab, dom_rand_lab1


def kernel(sfeat, tfeat):
    B, C, D = _B, _C, _D
    tem_rand_lab, dom_rand_lab1 = _labels()
    dom_set = jnp.asarray(
        np.array(list(product(*[[0, 1]] * C)), dtype=np.int32))
    bits = jnp.take(dom_set, dom_rand_lab1, axis=0)
    mask = bits.astype(jnp.float32)[:, :, None]

    nb = B // _BS
    out = pl.pallas_call(
        _select_kernel,
        grid=(nb,),
        in_specs=[
            pl.BlockSpec((_BS, C, 1), lambda b: (b, 0, 0)),
            pl.BlockSpec((_BS, C, D), lambda b: (b, 0, 0)),
            pl.BlockSpec((_BS, C, D), lambda b: (b, 0, 0)),
        ],
        out_specs=pl.BlockSpec((2, _BS, C, D), lambda b: (0, b, 0, 0)),
        out_shape=jax.ShapeDtypeStruct((2, B, C, D), sfeat.dtype),
    )(mask, sfeat, tfeat)
    feat = out.reshape(2 * B, C, D)

    dom_lab = jnp.concatenate([dom_rand_lab1, _DOM_LEN - 1 - dom_rand_lab1])
    tmp_lab = jnp.concatenate([tem_rand_lab, tem_rand_lab])
    dom_conf_lab = jnp.full((2 * B, _DOM_LEN), 1.0 / _DOM_LEN, jnp.float32)
    tmp_conf_lab = jnp.full((2 * B, _TMP_LEN), 1.0 / _TMP_LEN, jnp.float32)
    return (feat, dom_lab, dom_conf_lab, tmp_lab, tmp_conf_lab)
